# baseline (speedup 1.0000x reference)
import jax
import jax.numpy as jnp
from jax import lax
from jax.experimental import pallas as pl
from jax.experimental.pallas import tpu as pltpu
from jax.experimental.pallas import tpu_sc as plsc

N_FIELDS = 26
VOCAB = 100000
D_MODEL = 64
BATCH = 16384

NC, NS, L = 2, 16, 16
NW = NC * NS
CHUNK = BATCH // NW
IDX_W = 128
GPC = CHUNK // IDX_W

_mesh = plsc.VectorSubcoreMesh(core_axis_name="c", subcore_axis_name="s")


def _prep_body(xn_hbm, x1_hbm, vrow):
    wid = lax.axis_index("s") * NC + lax.axis_index("c")
    b0 = wid * CHUNK
    for f in range(N_FIELDS):
        pltpu.sync_copy(xn_hbm.at[f, pl.ds(b0, CHUNK)], vrow)
        off = f * VOCAB
        for k in range(CHUNK // L):
            vrow[pl.ds(k * L, L)] = vrow[pl.ds(k * L, L)] + off
        pltpu.sync_copy(vrow, x1_hbm.at[pl.ds(f * BATCH + b0, CHUNK)])


_prep = pl.kernel(
    _prep_body,
    out_type=jax.ShapeDtypeStruct((N_FIELDS * BATCH,), jnp.int32),
    mesh=_mesh,
    scratch_types=[pltpu.VMEM((CHUNK,), jnp.int32)],
    compiler_params=pltpu.CompilerParams(use_tc_tiling_on_sc=True),
)


def _gather_body(x1_hbm, tab_hbm, out_hbm, idx0, idx1, rows0, rows1,
                 gs0, gs1, ws0, ws1):
    wid = lax.axis_index("s") * NC + lax.axis_index("c")
    b0 = wid * CHUNK

    idxs = (idx0, idx1)
    bufs = (rows0, rows1)
    gsems = (gs0, gs1)
    wsems = (ws0, ws1)

    def load_idx(f, b):
        for q in range(GPC):
            pltpu.sync_copy(
                x1_hbm.at[pl.ds(f * BATCH + b0 + q * IDX_W, IDX_W)],
                idxs[b].at[q])

    def fire(b):
        for q in range(GPC):
            pltpu.async_copy(
                tab_hbm.at[idxs[b].at[q]],
                bufs[b].at[pl.ds(q * IDX_W, IDX_W)],
                gsems[b])

    def wait_full(b, sem):
        pltpu.make_async_copy(out_hbm.at[0, pl.ds(0, CHUNK)], bufs[b], sem).wait()

    def put(f, b):
        pltpu.async_copy(bufs[b], out_hbm.at[f, pl.ds(b0, CHUNK)], wsems[b])

    load_idx(0, 0)
    fire(0)

    def pair(p, _):
        f0 = 2 * p

        @pl.when(p >= 1)
        def _():
            wait_full(1, wsems[1])
        load_idx(f0 + 1, 1)
        fire(1)

        wait_full(0, gsems[0])
        put(f0, 0)
        wait_full(0, wsems[0])

        @pl.when(p < N_FIELDS // 2 - 1)
        def _():
            load_idx(f0 + 2, 0)
            fire(0)

        wait_full(1, gsems[1])
        put(f0 + 1, 1)
        return 0

    lax.fori_loop(0, N_FIELDS // 2, pair, 0)
    wait_full(1, wsems[1])


_gather = pl.kernel(
    _gather_body,
    out_type=jax.ShapeDtypeStruct((N_FIELDS, BATCH, D_MODEL), jnp.float32),
    mesh=_mesh,
    scratch_types=[
        pltpu.VMEM((GPC, IDX_W), jnp.int32),
        pltpu.VMEM((GPC, IDX_W), jnp.int32),
        pltpu.VMEM((CHUNK, D_MODEL), jnp.float32),
        pltpu.VMEM((CHUNK, D_MODEL), jnp.float32),
        pltpu.SemaphoreType.DMA,
        pltpu.SemaphoreType.DMA,
        pltpu.SemaphoreType.DMA,
        pltpu.SemaphoreType.DMA,
    ],
    compiler_params=pltpu.CompilerParams(use_tc_tiling_on_sc=False),
)


@jax.jit
def kernel(x_cat, tables):
    x1 = _prep(x_cat.T)
    tab = tables.reshape(N_FIELDS * VOCAB, D_MODEL)
    out = _gather(x1, tab)
    return out.transpose(1, 0, 2)
